# baseline (speedup 1.0000x reference)
import functools

import jax
import jax.numpy as jnp
import numpy as np
from jax import lax
from jax.experimental import pallas as pl
from jax.experimental.pallas import tpu as pltpu
from jax.experimental.pallas import tpu_sc as plsc

_B, _I, _H, _C = 8, 224, 224, 96
_D = _H * _C
_R = _B * _I
_NC, _NS = 2, 16
_NW = _NC * _NS
_RPW = _R // _NW
_K = 2
_NBUF = 2
_G = _RPW // _K


def _index_table():
    perm = jax.random.permutation(jax.random.key(1), _I)
    r = np.arange(_R)
    gidx = (r // _I) * _I + perm[r % _I]
    return gidx.astype(jnp.int32).reshape(_NW, _G, _K)


def _make_sc_shuffle():
    mesh = plsc.VectorSubcoreMesh(
        core_axis_name="c", subcore_axis_name="s",
        num_cores=_NC, num_subcores=_NS,
    )

    @functools.partial(
        pl.kernel,
        out_type=jax.ShapeDtypeStruct((_R, _D), jnp.float32),
        mesh=mesh,
        scratch_types=[
            pltpu.VMEM((_G, _K), jnp.int32),
            pltpu.VMEM((_K, _D), jnp.float32),
            pltpu.VMEM((_K, _D), jnp.float32),
            pltpu.SemaphoreType.DMA,
            pltpu.SemaphoreType.DMA,
            pltpu.SemaphoreType.DMA,
        ],
    )
    def shuffle(in_hbm, idx_hbm, out_hbm, idx_v, buf0, buf1,
                sem_g, sem_w0, sem_w1):
        wid = lax.axis_index("s") * _NC + lax.axis_index("c")
        base = wid * _RPW
        pltpu.sync_copy(idx_hbm.at[wid], idx_v)
        bufs = (buf0, buf1)
        sem_ws = (sem_w0, sem_w1)

        @pl.loop(0, _G, step=_NBUF)
        def _chunks(g0):
            for b in range(_NBUF):
                g = g0 + b
                row0 = base + g * _K

                @pl.when(g0 > 0)
                def _():
                    pltpu.make_async_copy(
                        bufs[b], out_hbm.at[pl.ds(row0, _K)], sem_ws[b]
                    ).wait()

                pltpu.async_copy(in_hbm.at[idx_v.at[g]], bufs[b], sem_g).wait()
                pltpu.async_copy(bufs[b], out_hbm.at[pl.ds(row0, _K)],
                                 sem_ws[b])

        for b in range(_NBUF):
            pltpu.make_async_copy(
                bufs[b], out_hbm.at[pl.ds(base, _K)], sem_ws[b]
            ).wait()

    return shuffle


def kernel(inputs):
    x2 = inputs.reshape(_R, _D)
    idx = _index_table()
    out2 = _make_sc_shuffle()(x2, idx)
    return out2.reshape(_B, _I, _H, _C)


# speedup vs baseline: 1.3160x; 1.3160x over previous
import functools

import jax
import jax.numpy as jnp
import numpy as np
from jax import lax
from jax.experimental import pallas as pl
from jax.experimental.pallas import tpu as pltpu
from jax.experimental.pallas import tpu_sc as plsc

_B, _I, _H, _C = 8, 224, 224, 96
_R = _B * _I
_NC, _NS = 2, 16
_NW = _NC * _NS
_RPW = _R // _NW
_IPAD = 64
_L = 16
_NBUF = 4


def _index_table():
    perm = jax.random.permutation(jax.random.key(1), _I)
    r = np.arange(_R)
    gidx = (r // _I) * _I + perm[r % _I]
    gidx = gidx.reshape(_NW, _RPW)
    pad = np.zeros((_NW, _IPAD - _RPW), dtype=gidx.dtype)
    return jnp.concatenate(
        [gidx, pad], axis=1
    ).astype(jnp.int32).reshape(_NW * _IPAD)


def _make_sc_shuffle():
    mesh = plsc.VectorSubcoreMesh(
        core_axis_name="c", subcore_axis_name="s",
        num_cores=_NC, num_subcores=_NS,
    )

    @functools.partial(
        pl.kernel,
        out_type=jax.ShapeDtypeStruct((_R, _H, _C), jnp.float32),
        mesh=mesh,
        compiler_params=pltpu.CompilerParams(
            use_tc_tiling_on_sc=True, needs_layout_passes=False
        ),
        scratch_types=[
            pltpu.VMEM((_IPAD,), jnp.int32),
            pltpu.VMEM((_NBUF, _H, _C), jnp.float32),
            pltpu.SemaphoreType.DMA,
            pltpu.SemaphoreType.DMA,
            pltpu.SemaphoreType.DMA,
            pltpu.SemaphoreType.DMA,
            pltpu.SemaphoreType.DMA,
        ],
    )
    def shuffle(in_hbm, idx_hbm, out_hbm, idx_v, buf,
                sem_g, sem_w0, sem_w1, sem_w2, sem_w3):
        wid = lax.axis_index("s") * _NC + lax.axis_index("c")
        base = wid * _RPW
        pltpu.sync_copy(idx_hbm.at[pl.ds(wid * _IPAD, _IPAD)], idx_v)
        sem_ws = (sem_w0, sem_w1, sem_w2, sem_w3)
        lanes = lax.iota(jnp.int32, _L)

        @pl.loop(0, _RPW, step=_NBUF)
        def _slabs(g0):
            for b in range(_NBUF):
                g = g0 + b
                dst = base + g

                @pl.when(g0 > 0)
                def _():
                    pltpu.make_async_copy(
                        buf.at[b], out_hbm.at[dst], sem_ws[b]
                    ).wait()

                word = pl.multiple_of((g // _L) * _L, _L)
                vec = idx_v[pl.ds(word, _L)]
                sel = jnp.where(lanes == g % _L, vec, 0)
                src = lax.reduce_max(sel, axes=(0,))

                pltpu.async_copy(in_hbm.at[src], buf.at[b], sem_g).wait()
                pltpu.async_copy(buf.at[b], out_hbm.at[dst], sem_ws[b])

        for b in range(_NBUF):
            pltpu.make_async_copy(
                buf.at[b], out_hbm.at[base + b], sem_ws[b]
            ).wait()

    return shuffle


def kernel(inputs):
    x3 = inputs.reshape(_R, _H, _C)
    idx = _index_table()
    out3 = _make_sc_shuffle()(x3, idx)
    return out3.reshape(_B, _I, _H, _C)


# speedup vs baseline: 8.5991x; 6.5343x over previous
import functools

import jax
import jax.numpy as jnp
import numpy as np
from jax import lax
from jax.experimental import pallas as pl
from jax.experimental.pallas import tpu as pltpu
from jax.experimental.pallas import tpu_sc as plsc

_B, _I, _H, _C = 8, 224, 224, 96
_R = _B * _I
_NC, _NS = 2, 16
_NW = _NC * _NS
_RPW = _R // _NW
_IPAD = 64
_L = 16
_NBUF = 4

_PERM = np.array([
    183, 138, 166, 19, 76, 158, 219, 118, 143, 54, 189, 149, 90, 30, 7,
    96, 139, 155, 131, 121, 115, 6, 35, 23, 58, 128, 16, 21, 194, 213,
    156, 220, 77, 154, 160, 94, 116, 61, 38, 3, 185, 105, 132, 81, 26,
    32, 64, 37, 56, 51, 2, 193, 122, 63, 133, 52, 20, 89, 202, 95, 44,
    47, 123, 79, 84, 222, 144, 157, 135, 50, 140, 78, 179, 72, 163, 191,
    83, 42, 62, 152, 69, 53, 223, 148, 172, 215, 0, 201, 145, 8, 208,
    203, 167, 169, 159, 109, 181, 22, 178, 13, 29, 99, 110, 34, 70, 175,
    18, 103, 196, 141, 86, 142, 75, 198, 187, 206, 91, 111, 24, 113, 1,
    65, 48, 5, 45, 199, 165, 150, 49, 173, 214, 33, 216, 74, 55, 182,
    136, 60, 204, 119, 57, 124, 27, 112, 129, 209, 151, 10, 134, 192,
    186, 93, 176, 161, 68, 146, 15, 217, 73, 40, 210, 67, 88, 102, 107,
    66, 80, 100, 120, 211, 147, 71, 207, 17, 59, 184, 98, 108, 114, 36,
    125, 101, 218, 180, 92, 171, 153, 28, 46, 9, 104, 200, 117, 221, 4,
    177, 170, 190, 130, 12, 168, 195, 188, 87, 85, 212, 14, 174, 82, 31,
    106, 127, 162, 126, 164, 97, 41, 137, 197, 25, 43, 39, 11, 205,
], dtype=np.int32)


def _index_table() -> np.ndarray:
    r = np.arange(_R)
    gidx = ((r // _I) * _I + _PERM[r % _I]).astype(np.int32)
    gidx = gidx.reshape(_NW, _RPW)
    pad = np.zeros((_NW, _IPAD - _RPW), dtype=np.int32)
    return np.concatenate([gidx, pad], axis=1).reshape(_NW * _IPAD)


def _make_sc_shuffle():
    mesh = plsc.VectorSubcoreMesh(
        core_axis_name="c", subcore_axis_name="s",
        num_cores=_NC, num_subcores=_NS,
    )

    @functools.partial(
        pl.kernel,
        out_type=jax.ShapeDtypeStruct((_R, _C, _H), jnp.float32),
        mesh=mesh,
        compiler_params=pltpu.CompilerParams(
            use_tc_tiling_on_sc=True, needs_layout_passes=False
        ),
        scratch_types=[
            pltpu.VMEM((_IPAD,), jnp.int32),
            pltpu.VMEM((_NBUF, _C, _H), jnp.float32),
            pltpu.SemaphoreType.DMA,
            pltpu.SemaphoreType.DMA,
            pltpu.SemaphoreType.DMA,
            pltpu.SemaphoreType.DMA,
            pltpu.SemaphoreType.DMA,
        ],
    )
    def shuffle(in_hbm, idx_hbm, out_hbm, idx_v, buf,
                sem_g, sem_w0, sem_w1, sem_w2, sem_w3):
        wid = lax.axis_index("s") * _NC + lax.axis_index("c")
        base = wid * _RPW
        pltpu.sync_copy(idx_hbm.at[pl.ds(wid * _IPAD, _IPAD)], idx_v)
        sem_ws = (sem_w0, sem_w1, sem_w2, sem_w3)
        lanes = lax.iota(jnp.int32, _L)

        @pl.loop(0, _RPW, step=_NBUF)
        def _slabs(g0):
            for b in range(_NBUF):
                g = g0 + b
                dst = base + g

                @pl.when(g0 > 0)
                def _():
                    pltpu.make_async_copy(
                        buf.at[b], out_hbm.at[dst], sem_ws[b]
                    ).wait()

                word = pl.multiple_of((g // _L) * _L, _L)
                vec = idx_v[pl.ds(word, _L)]
                sel = jnp.where(lanes == g % _L, vec, 0)
                src = lax.reduce_max(sel, axes=(0,))

                pltpu.async_copy(in_hbm.at[src], buf.at[b], sem_g).wait()
                pltpu.async_copy(buf.at[b], out_hbm.at[dst], sem_ws[b])

        for b in range(_NBUF):
            pltpu.make_async_copy(
                buf.at[b], out_hbm.at[base + b], sem_ws[b]
            ).wait()

    return shuffle


def kernel(inputs):
    x3 = jnp.transpose(inputs, (0, 1, 3, 2)).reshape(_R, _C, _H)
    idx = jnp.asarray(_index_table())
    out3 = _make_sc_shuffle()(x3, idx)
    return jnp.transpose(out3.reshape(_B, _I, _C, _H), (0, 1, 3, 2))


# speedup vs baseline: 9.1091x; 1.0593x over previous
import functools

import jax
import jax.numpy as jnp
import numpy as np
from jax import lax
from jax.experimental import pallas as pl
from jax.experimental.pallas import tpu as pltpu
from jax.experimental.pallas import tpu_sc as plsc

_B, _I, _H, _C = 8, 224, 224, 96
_R = _B * _I
_NC, _NS = 2, 16
_NW = _NC * _NS
_RPW = _R // _NW
_IPAD = 64
_L = 16
_NBUF = 4

_PERM = np.array([
    183, 138, 166, 19, 76, 158, 219, 118, 143, 54, 189, 149, 90, 30, 7,
    96, 139, 155, 131, 121, 115, 6, 35, 23, 58, 128, 16, 21, 194, 213,
    156, 220, 77, 154, 160, 94, 116, 61, 38, 3, 185, 105, 132, 81, 26,
    32, 64, 37, 56, 51, 2, 193, 122, 63, 133, 52, 20, 89, 202, 95, 44,
    47, 123, 79, 84, 222, 144, 157, 135, 50, 140, 78, 179, 72, 163, 191,
    83, 42, 62, 152, 69, 53, 223, 148, 172, 215, 0, 201, 145, 8, 208,
    203, 167, 169, 159, 109, 181, 22, 178, 13, 29, 99, 110, 34, 70, 175,
    18, 103, 196, 141, 86, 142, 75, 198, 187, 206, 91, 111, 24, 113, 1,
    65, 48, 5, 45, 199, 165, 150, 49, 173, 214, 33, 216, 74, 55, 182,
    136, 60, 204, 119, 57, 124, 27, 112, 129, 209, 151, 10, 134, 192,
    186, 93, 176, 161, 68, 146, 15, 217, 73, 40, 210, 67, 88, 102, 107,
    66, 80, 100, 120, 211, 147, 71, 207, 17, 59, 184, 98, 108, 114, 36,
    125, 101, 218, 180, 92, 171, 153, 28, 46, 9, 104, 200, 117, 221, 4,
    177, 170, 190, 130, 12, 168, 195, 188, 87, 85, 212, 14, 174, 82, 31,
    106, 127, 162, 126, 164, 97, 41, 137, 197, 25, 43, 39, 11, 205,
], dtype=np.int32)


def _index_table() -> np.ndarray:
    r = np.arange(_R)
    gidx = ((r // _I) * _I + _PERM[r % _I]).astype(np.int32)
    gidx = gidx.reshape(_NW, _RPW)
    pad = np.zeros((_NW, _IPAD - _RPW), dtype=np.int32)
    return np.concatenate([gidx, pad], axis=1).reshape(_NW * _IPAD)


def _make_sc_shuffle():
    mesh = plsc.VectorSubcoreMesh(
        core_axis_name="c", subcore_axis_name="s",
        num_cores=_NC, num_subcores=_NS,
    )

    @functools.partial(
        pl.kernel,
        out_type=jax.ShapeDtypeStruct((_R, _C, _H), jnp.float32),
        mesh=mesh,
        compiler_params=pltpu.CompilerParams(
            use_tc_tiling_on_sc=True, needs_layout_passes=False
        ),
        scratch_types=[
            pltpu.VMEM((_IPAD,), jnp.int32),
            pltpu.VMEM((_NBUF, _C, _H), jnp.float32),
            pltpu.SemaphoreType.DMA,
            pltpu.SemaphoreType.DMA,
            pltpu.SemaphoreType.DMA,
            pltpu.SemaphoreType.DMA,
            pltpu.SemaphoreType.DMA,
            pltpu.SemaphoreType.DMA,
            pltpu.SemaphoreType.DMA,
            pltpu.SemaphoreType.DMA,
        ],
    )
    def shuffle(in_hbm, idx_hbm, out_hbm, idx_v, buf,
                sem_g0, sem_g1, sem_g2, sem_g3,
                sem_w0, sem_w1, sem_w2, sem_w3):
        wid = lax.axis_index("s") * _NC + lax.axis_index("c")
        base = wid * _RPW
        pltpu.sync_copy(idx_hbm.at[pl.ds(wid * _IPAD, _IPAD)], idx_v)
        sem_gs = (sem_g0, sem_g1, sem_g2, sem_g3)
        sem_ws = (sem_w0, sem_w1, sem_w2, sem_w3)
        lanes = lax.iota(jnp.int32, _L)

        def _fire_gather(g, b):
            word = pl.multiple_of((g // _L) * _L, _L)
            vec = idx_v[pl.ds(word, _L)]
            sel = jnp.where(lanes == g % _L, vec, 0)
            src = lax.reduce_max(sel, axes=(0,))
            pltpu.async_copy(in_hbm.at[src], buf.at[b], sem_gs[b])

        for b in range(2):
            _fire_gather(b, b)

        @pl.loop(0, _RPW, step=_NBUF)
        def _group(g0):
            for b in range(_NBUF):
                g = g0 + b
                ng = g + 2
                nb = (b + 2) % _NBUF

                @pl.when(ng < _RPW)
                def _():
                    @pl.when(ng >= _NBUF)
                    def _():
                        pltpu.make_async_copy(
                            buf.at[nb], out_hbm.at[base], sem_ws[nb]
                        ).wait()

                    _fire_gather(ng, nb)

                pltpu.make_async_copy(
                    in_hbm.at[base], buf.at[b], sem_gs[b]
                ).wait()
                pltpu.async_copy(buf.at[b], out_hbm.at[base + g], sem_ws[b])

        for b in range(_NBUF):
            pltpu.make_async_copy(
                buf.at[b], out_hbm.at[base + b], sem_ws[b]
            ).wait()

    return shuffle


def kernel(inputs):
    x3 = jnp.transpose(inputs, (0, 1, 3, 2)).reshape(_R, _C, _H)
    idx = jnp.asarray(_index_table())
    out3 = _make_sc_shuffle()(x3, idx)
    return jnp.transpose(out3.reshape(_B, _I, _C, _H), (0, 1, 3, 2))
